# baseline (speedup 1.0000x reference)
import functools

import jax
import jax.numpy as jnp
from jax import lax
from jax.experimental import pallas as pl
from jax.experimental.pallas import tpu as pltpu
from jax.experimental.pallas import tpu_sc as plsc

N = 10000
D = 128
E = 320000
EP = 327680
G = 128
NCLS = 10

NC, NS = 2, 16
NW = NC * NS
C = 80

NP = 10240
NPS = NP // NS
PAD_NODE = N
DW = 16

_mesh = plsc.VectorSubcoreMesh(core_axis_name="c", subcore_axis_name="s")


NCHUNK_DEG = EP // NW // C
NB_DEG = 8
NK_DEG = NCHUNK_DEG // NB_DEG


@functools.partial(
    pl.kernel,
    out_type=jax.ShapeDtypeStruct((NC * NP, DW), jnp.float32),
    mesh=_mesh,
    scratch_types=[
        pltpu.VMEM((NCHUNK_DEG, C), jnp.int32),
        pltpu.VMEM((C, DW), jnp.float32),
        pltpu.VMEM((NPS, DW), jnp.float32),
        pltpu.VMEM_SHARED((NP, DW), jnp.float32),
        [pltpu.SemaphoreType.DMA] * NB_DEG,
    ],
    compiler_params=pltpu.CompilerParams(use_tc_tiling_on_sc=False),
)
def _sc_degree(dst_hbm, zeros_hbm, ones_hbm, out_hbm, didx, ones, bounce, acc,
               sems):
    c = lax.axis_index("c")
    s = lax.axis_index("s")
    wid = s * NC + c
    pltpu.sync_copy(dst_hbm.at[pl.ds(wid * NCHUNK_DEG, NCHUNK_DEG)], didx)
    pltpu.sync_copy(ones_hbm, ones)
    pltpu.sync_copy(zeros_hbm, bounce)
    pltpu.sync_copy(bounce, acc.at[pl.ds(s * NPS, NPS)])
    plsc.subcore_barrier()

    def body(k, carry):
        for b in range(NB_DEG):
            j = k * NB_DEG + b

            @pl.when(k > 0)
            def _():
                pltpu.make_async_copy(ones, acc.at[didx.at[j - NB_DEG]],
                                      sems[b]).wait()

            pltpu.async_copy(ones, acc.at[didx.at[j]], sems[b], add=True)
        return carry

    lax.fori_loop(0, NK_DEG, body, 0)
    for b in range(NB_DEG):
        j = (NK_DEG - 1) * NB_DEG + b
        pltpu.make_async_copy(ones, acc.at[didx.at[j]], sems[b]).wait()
    plsc.subcore_barrier()
    pltpu.sync_copy(acc.at[pl.ds(s * NPS, NPS)], bounce)
    pltpu.sync_copy(bounce, out_hbm.at[pl.ds(c * NP + s * NPS, NPS)])


NCHUNK = EP // NW // C
NBG = 4
NGRP = NCHUNK // NBG
NPAIR = NGRP // 2
NQ = NPS // C


@functools.partial(
    pl.kernel,
    out_type=jax.ShapeDtypeStruct((NC * NP, D), jnp.float32),
    mesh=_mesh,
    scratch_types=[
        [pltpu.VMEM((NBG, C), jnp.int32)] * 2,
        [pltpu.VMEM((NBG, C), jnp.int32)] * 2,
        [pltpu.VMEM((C, D), jnp.float32)] * NBG,
        pltpu.VMEM_SHARED((NP, D), jnp.float32),
        [pltpu.SemaphoreType.DMA] * 2,
        [pltpu.SemaphoreType.DMA] * NBG,
        [pltpu.SemaphoreType.DMA] * NBG,
    ],
)
def _sc_prop(u_hbm, src_hbm, dst_hbm, zeros_hbm, out_hbm,
             sidx, didx, rows, acc, isem, gsem, ssem):
    c = lax.axis_index("c")
    s = lax.axis_index("s")
    w = c * NS + s
    gbase = w * NGRP

    def idx_load(g, p):
        pltpu.async_copy(src_hbm.at[pl.ds((gbase + g) * NBG, NBG)], sidx[p],
                         isem[p])
        pltpu.async_copy(dst_hbm.at[pl.ds((gbase + g) * NBG, NBG)], didx[p],
                         isem[p])

    def idx_wait(g, p):
        pltpu.make_async_copy(src_hbm.at[pl.ds((gbase + g) * NBG, NBG)],
                              sidx[p], isem[p]).wait()
        pltpu.make_async_copy(dst_hbm.at[pl.ds((gbase + g) * NBG, NBG)],
                              didx[p], isem[p]).wait()

    def gather(b, p):
        pltpu.async_copy(u_hbm.at[sidx[p].at[b]], rows[b], gsem[b])

    def gather_wait(b, p):
        pltpu.make_async_copy(u_hbm.at[sidx[p].at[b]], rows[b],
                              gsem[b]).wait()

    def scatter(b, p):
        pltpu.async_copy(rows[b], acc.at[didx[p].at[b]], ssem[b], add=True)

    def scatter_wait(b, p):
        pltpu.make_async_copy(rows[b], acc.at[didx[p].at[b]], ssem[b]).wait()

    idx_load(0, 0)
    pltpu.sync_copy(zeros_hbm, rows[0])
    zdescs = [
        pltpu.async_copy(rows[0], acc.at[pl.ds(s * NPS + q * C, C)],
                         ssem[q % NBG])
        for q in range(NQ)
    ]
    for dsc in zdescs:
        dsc.wait()
    plsc.subcore_barrier()
    idx_wait(0, 0)
    idx_load(1, 1)
    for b in range(NBG):
        gather(b, 0)

    def body(p, carry):
        for b in range(NBG):
            gather_wait(b, 0)
            scatter(b, 0)

        @pl.when(p < NPAIR - 1)
        def _():
            idx_wait(2 * p + 1, 1)

        for b in range(NBG):
            scatter_wait(b, 0)

            @pl.when(p < NPAIR - 1)
            def _():
                gather(b, 1)

        @pl.when(p < NPAIR - 1)
        def _():
            idx_load(2 * p + 2, 0)

        @pl.when(p < NPAIR - 1)
        def _():
            for b in range(NBG):
                gather_wait(b, 1)
                scatter(b, 1)
            idx_wait(2 * p + 2, 0)
            for b in range(NBG):
                scatter_wait(b, 1)
                gather(b, 0)
            idx_load(2 * p + 3, 1)

        return carry

    lax.fori_loop(0, NPAIR, body, 0)
    idx_wait(2 * NPAIR - 1, 1)
    for b in range(NBG):
        gather(b, 1)
    for b in range(NBG):
        gather_wait(b, 1)
        scatter(b, 1)
    for b in range(NBG):
        scatter_wait(b, 1)
    plsc.subcore_barrier()
    odescs = []
    for q in range(NQ):
        if q >= NBG:
            odescs[q - NBG].wait()
        pltpu.sync_copy(acc.at[pl.ds(s * NPS + q * C, C)], rows[q % NBG])
        odescs.append(pltpu.async_copy(
            rows[q % NBG],
            out_hbm.at[pl.ds(c * NP + s * NPS + q * C, C)], gsem[q % NBG]))
    for dsc in odescs[NQ - NBG:]:
        dsc.wait()


R = 400
NBLK = N // R


def _tc0_body(x_ref, w_ref, deg_ref, u_ref, dis_ref):
    d = deg_ref[:, 0:1] + deg_ref[:, 1:2] + 1.0
    dis = lax.rsqrt(d)
    dis_ref[...] = dis
    u_ref[...] = jnp.dot(x_ref[...], w_ref[...],
                         preferred_element_type=jnp.float32) * dis


_tc0 = pl.pallas_call(
    _tc0_body,
    grid=(NBLK,),
    in_specs=[
        pl.BlockSpec((R, D), lambda i: (i, 0)),
        pl.BlockSpec((D, D), lambda i: (0, 0)),
        pl.BlockSpec((R, 2), lambda i: (i, 0)),
    ],
    out_specs=[
        pl.BlockSpec((R, D), lambda i: (i, 0)),
        pl.BlockSpec((R, 1), lambda i: (i, 0)),
    ],
    out_shape=[
        jax.ShapeDtypeStruct((N, D), jnp.float32),
        jax.ShapeDtypeStruct((N, 1), jnp.float32),
    ],
)


def _tc_mid_body(a_ref, u_ref, dis_ref, b_ref, w_ref, h_ref, un_ref):
    dis = dis_ref[...]
    agg = (a_ref[0] + a_ref[1] + u_ref[...]) * dis + b_ref[...]
    h = jnp.maximum(agg, 0.0)
    h_ref[...] = h
    un_ref[...] = jnp.dot(h, w_ref[...],
                          preferred_element_type=jnp.float32) * dis


_tc_mid = pl.pallas_call(
    _tc_mid_body,
    grid=(NBLK,),
    in_specs=[
        pl.BlockSpec((NC, R, D), lambda i: (0, i, 0)),
        pl.BlockSpec((R, D), lambda i: (i, 0)),
        pl.BlockSpec((R, 1), lambda i: (i, 0)),
        pl.BlockSpec((1, D), lambda i: (0, 0)),
        pl.BlockSpec((D, D), lambda i: (0, 0)),
    ],
    out_specs=[
        pl.BlockSpec((R, D), lambda i: (i, 0)),
        pl.BlockSpec((R, D), lambda i: (i, 0)),
    ],
    out_shape=[
        jax.ShapeDtypeStruct((N, D), jnp.float32),
        jax.ShapeDtypeStruct((N, D), jnp.float32),
    ],
)


def _tc_final_body(a_ref, u_ref, dis_ref, b_ref, batch_ref, wl_ref, bl_ref,
                   h_ref, z_ref, p_scr):
    i = pl.program_id(0)
    agg = (a_ref[0] + a_ref[1] + u_ref[...]) * dis_ref[...] + b_ref[...]
    h = jnp.maximum(agg, 0.0)
    h_ref[...] = h
    onehot = (lax.broadcasted_iota(jnp.int32, (G, R), 0)
              == batch_ref[0]).astype(jnp.float32)
    part = jax.lax.dot_general(onehot, h, (((1,), (0,)), ((), ())),
                               preferred_element_type=jnp.float32)

    @pl.when(i == 0)
    def _():
        p_scr[...] = jnp.zeros_like(p_scr)

    p_scr[...] += part

    @pl.when(i == NBLK - 1)
    def _():
        z_ref[...] = jnp.dot(p_scr[...], wl_ref[...],
                             preferred_element_type=jnp.float32) + bl_ref[...]


_tc_final = pl.pallas_call(
    _tc_final_body,
    grid=(NBLK,),
    in_specs=[
        pl.BlockSpec((NC, R, D), lambda i: (0, i, 0)),
        pl.BlockSpec((R, D), lambda i: (i, 0)),
        pl.BlockSpec((R, 1), lambda i: (i, 0)),
        pl.BlockSpec((1, D), lambda i: (0, 0)),
        pl.BlockSpec((1, 1, R), lambda i: (i, 0, 0)),
        pl.BlockSpec((D, NCLS), lambda i: (0, 0)),
        pl.BlockSpec((1, NCLS), lambda i: (0, 0)),
    ],
    out_specs=[
        pl.BlockSpec((R, D), lambda i: (i, 0)),
        pl.BlockSpec((G, NCLS), lambda i: (0, 0)),
    ],
    out_shape=[
        jax.ShapeDtypeStruct((N, D), jnp.float32),
        jax.ShapeDtypeStruct((G, NCLS), jnp.float32),
    ],
    scratch_shapes=[pltpu.VMEM((G, D), jnp.float32)],
)


def kernel(x, edge_index, batch, W1, b1, W2, b2, W3, b3, W4, b4, Wl, bl):
    src = edge_index[0].astype(jnp.int32)
    dst = edge_index[1].astype(jnp.int32)
    pad = EP - E
    srcp = jnp.concatenate([src, jnp.zeros((pad,), jnp.int32)])
    dstp = jnp.concatenate([dst, jnp.full((pad,), PAD_NODE, jnp.int32)])
    src2 = srcp.reshape(EP // C, C)
    dst2 = dstp.reshape(EP // C, C)
    batch_row = batch.astype(jnp.int32).reshape(NBLK, 1, R)

    zrows = jnp.zeros((C, D), jnp.float32)
    zdeg = jnp.zeros((NPS, DW), jnp.float32)
    ones = jnp.ones((C, DW), jnp.float32)

    deg = _sc_degree(dst2, zdeg, ones).reshape(NC, NP, DW)
    degT = deg[:, :N, 0].T

    def prop(u):
        a = _sc_prop(u, src2, dst2, zrows)
        return a.reshape(NC, NP, D)

    u1, dis = _tc0(x, W1, degT)
    a1 = prop(u1)
    h1, u2 = _tc_mid(a1, u1, dis, b1.reshape(1, D), W2)
    a2 = prop(u2)
    h2, u3 = _tc_mid(a2, u2, dis, b2.reshape(1, D), W3)
    a3 = prop(u3)
    h3, u4 = _tc_mid(a3, u3, dis, b3.reshape(1, D), W4)
    a4 = prop(u4)
    h4, z = _tc_final(a4, u4, dis, b4.reshape(1, D), batch_row, Wl,
                      bl.reshape(1, NCLS))
    return (h1, h2, h3, h4, z)


# speedup vs baseline: 1.0153x; 1.0153x over previous
import functools

import jax
import jax.numpy as jnp
from jax import lax
from jax.experimental import pallas as pl
from jax.experimental.pallas import tpu as pltpu
from jax.experimental.pallas import tpu_sc as plsc

N = 10000
D = 128
E = 320000
EP = 327680
G = 128
NCLS = 10

NC, NS = 2, 16
NW = NC * NS
C = 80

NP = 10240
NPS = NP // NS
PAD_NODE = N
DW = 16

_mesh = plsc.VectorSubcoreMesh(core_axis_name="c", subcore_axis_name="s")


NCHUNK_DEG = EP // NW // C
NB_DEG = 8
NK_DEG = NCHUNK_DEG // NB_DEG


@functools.partial(
    pl.kernel,
    out_type=jax.ShapeDtypeStruct((NC * NP, DW), jnp.float32),
    mesh=_mesh,
    scratch_types=[
        pltpu.VMEM((NCHUNK_DEG, C), jnp.int32),
        pltpu.VMEM((C, DW), jnp.float32),
        pltpu.VMEM((NPS, DW), jnp.float32),
        pltpu.VMEM_SHARED((NP, DW), jnp.float32),
        [pltpu.SemaphoreType.DMA] * NB_DEG,
    ],
    compiler_params=pltpu.CompilerParams(use_tc_tiling_on_sc=False),
)
def _sc_degree(dst_hbm, zeros_hbm, ones_hbm, out_hbm, didx, ones, bounce, acc,
               sems):
    c = lax.axis_index("c")
    s = lax.axis_index("s")
    wid = s * NC + c
    pltpu.sync_copy(dst_hbm.at[pl.ds(wid * NCHUNK_DEG, NCHUNK_DEG)], didx)
    pltpu.sync_copy(ones_hbm, ones)
    pltpu.sync_copy(zeros_hbm, bounce)
    pltpu.sync_copy(bounce, acc.at[pl.ds(s * NPS, NPS)])
    plsc.subcore_barrier()

    def body(k, carry):
        for b in range(NB_DEG):
            j = k * NB_DEG + b

            @pl.when(k > 0)
            def _():
                pltpu.make_async_copy(ones, acc.at[didx.at[j - NB_DEG]],
                                      sems[b]).wait()

            pltpu.async_copy(ones, acc.at[didx.at[j]], sems[b], add=True)
        return carry

    lax.fori_loop(0, NK_DEG, body, 0)
    for b in range(NB_DEG):
        j = (NK_DEG - 1) * NB_DEG + b
        pltpu.make_async_copy(ones, acc.at[didx.at[j]], sems[b]).wait()
    plsc.subcore_barrier()
    pltpu.sync_copy(acc.at[pl.ds(s * NPS, NPS)], bounce)
    pltpu.sync_copy(bounce, out_hbm.at[pl.ds(c * NP + s * NPS, NPS)])


NCHUNK = EP // NW // C
NBG = 4
NGRP = NCHUNK // NBG
NPAIR = NGRP // 2
NQ = NPS // C


@functools.partial(
    pl.kernel,
    out_type=jax.ShapeDtypeStruct((NC * NP, D), jnp.float32),
    mesh=_mesh,
    scratch_types=[
        [pltpu.VMEM((NBG, C), jnp.int32)] * 2,
        [pltpu.VMEM((NBG, C), jnp.int32)] * 2,
        [pltpu.VMEM((C, D), jnp.float32)] * NBG,
        pltpu.VMEM_SHARED((NP, D), jnp.float32),
        [pltpu.SemaphoreType.DMA] * 2,
        [pltpu.SemaphoreType.DMA] * NBG,
        [pltpu.SemaphoreType.DMA] * NBG,
    ],
)
def _sc_prop(u_hbm, src_hbm, dst_hbm, zeros_hbm, out_hbm,
             sidx, didx, rows, acc, isem, gsem, ssem):
    c = lax.axis_index("c")
    s = lax.axis_index("s")
    w = c * NS + s
    gbase = w * NGRP

    def idx_load(g, p):
        pltpu.async_copy(src_hbm.at[pl.ds((gbase + g) * NBG, NBG)], sidx[p],
                         isem[p])
        pltpu.async_copy(dst_hbm.at[pl.ds((gbase + g) * NBG, NBG)], didx[p],
                         isem[p])

    def idx_wait(g, p):
        pltpu.make_async_copy(src_hbm.at[pl.ds((gbase + g) * NBG, NBG)],
                              sidx[p], isem[p]).wait()
        pltpu.make_async_copy(dst_hbm.at[pl.ds((gbase + g) * NBG, NBG)],
                              didx[p], isem[p]).wait()

    def gather(b, p):
        pltpu.async_copy(u_hbm.at[sidx[p].at[b]], rows[b], gsem[b])

    def gather_wait(b, p):
        pltpu.make_async_copy(u_hbm.at[sidx[p].at[b]], rows[b],
                              gsem[b]).wait()

    def scatter(b, p):
        pltpu.async_copy(rows[b], acc.at[didx[p].at[b]], ssem[b], add=True)

    def scatter_wait(b, p):
        pltpu.make_async_copy(rows[b], acc.at[didx[p].at[b]], ssem[b]).wait()

    idx_load(0, 0)
    pltpu.sync_copy(zeros_hbm, rows[0])
    zdescs = [
        pltpu.async_copy(rows[0], acc.at[pl.ds(s * NPS + q * C, C)],
                         ssem[q % NBG])
        for q in range(NQ)
    ]
    for dsc in zdescs:
        dsc.wait()
    plsc.subcore_barrier()
    idx_wait(0, 0)
    idx_load(1, 1)
    for b in range(NBG):
        gather(b, 0)

    def body(p, carry):
        for b in range(NBG):
            gather_wait(b, 0)
            scatter(b, 0)

        @pl.when(p < NPAIR - 1)
        def _():
            idx_wait(2 * p + 1, 1)

        for b in range(NBG):
            scatter_wait(b, 0)

            @pl.when(p < NPAIR - 1)
            def _():
                gather(b, 1)

        @pl.when(p < NPAIR - 1)
        def _():
            idx_load(2 * p + 2, 0)

        @pl.when(p < NPAIR - 1)
        def _():
            for b in range(NBG):
                gather_wait(b, 1)
                scatter(b, 1)
            idx_wait(2 * p + 2, 0)
            for b in range(NBG):
                scatter_wait(b, 1)
                gather(b, 0)
            idx_load(2 * p + 3, 1)

        return carry

    lax.fori_loop(0, NPAIR, body, 0)
    idx_wait(2 * NPAIR - 1, 1)
    for b in range(NBG):
        gather(b, 1)
    for b in range(NBG):
        gather_wait(b, 1)
        scatter(b, 1)
    for b in range(NBG):
        scatter_wait(b, 1)
    plsc.subcore_barrier()
    odescs = []
    for q in range(NQ):
        if q >= NBG:
            odescs[q - NBG].wait()
        pltpu.sync_copy(acc.at[pl.ds(s * NPS + q * C, C)], rows[q % NBG])
        odescs.append(pltpu.async_copy(
            rows[q % NBG],
            out_hbm.at[pl.ds(c * NP + s * NPS + q * C, C)], gsem[q % NBG]))
    for dsc in odescs[NQ - NBG:]:
        dsc.wait()


R = 400
NBLK = N // R


def _tc0_body(x_ref, w_ref, deg_ref, u_ref, dis_ref):
    d = deg_ref[:, 0:1] + deg_ref[:, 1:2] + 1.0
    dis = lax.rsqrt(d)
    dis_ref[...] = dis
    u_ref[...] = jnp.dot(x_ref[...], w_ref[...],
                         preferred_element_type=jnp.float32) * dis


_tc0 = pl.pallas_call(
    _tc0_body,
    grid=(NBLK,),
    in_specs=[
        pl.BlockSpec((R, D), lambda i: (i, 0)),
        pl.BlockSpec((D, D), lambda i: (0, 0)),
        pl.BlockSpec((R, 2), lambda i: (i, 0)),
    ],
    out_specs=[
        pl.BlockSpec((R, D), lambda i: (i, 0)),
        pl.BlockSpec((R, 1), lambda i: (i, 0)),
    ],
    out_shape=[
        jax.ShapeDtypeStruct((N, D), jnp.float32),
        jax.ShapeDtypeStruct((N, 1), jnp.float32),
    ],
)


def _tc_mid_body(a_ref, u_ref, dis_ref, b_ref, w_ref, h_ref, un_ref):
    dis = dis_ref[...]
    agg = (a_ref[0] + a_ref[1] + u_ref[...]) * dis + b_ref[...]
    h = jnp.maximum(agg, 0.0)
    h_ref[...] = h
    un_ref[...] = jnp.dot(h, w_ref[...],
                          preferred_element_type=jnp.float32) * dis


_tc_mid = pl.pallas_call(
    _tc_mid_body,
    grid=(NBLK,),
    in_specs=[
        pl.BlockSpec((NC, R, D), lambda i: (0, i, 0)),
        pl.BlockSpec((R, D), lambda i: (i, 0)),
        pl.BlockSpec((R, 1), lambda i: (i, 0)),
        pl.BlockSpec((1, D), lambda i: (0, 0)),
        pl.BlockSpec((D, D), lambda i: (0, 0)),
    ],
    out_specs=[
        pl.BlockSpec((R, D), lambda i: (i, 0)),
        pl.BlockSpec((R, D), lambda i: (i, 0)),
    ],
    out_shape=[
        jax.ShapeDtypeStruct((N, D), jnp.float32),
        jax.ShapeDtypeStruct((N, D), jnp.float32),
    ],
)


def _tc_final_body(a_ref, u_ref, dis_ref, b_ref, batch_ref, wl_ref, bl_ref,
                   h_ref, z_ref, p_scr):
    i = pl.program_id(0)
    agg = (a_ref[0] + a_ref[1] + u_ref[...]) * dis_ref[...] + b_ref[...]
    h = jnp.maximum(agg, 0.0)
    h_ref[...] = h
    onehot = (lax.broadcasted_iota(jnp.int32, (G, R), 0)
              == batch_ref[0]).astype(jnp.float32)
    part = jax.lax.dot_general(onehot, h, (((1,), (0,)), ((), ())),
                               preferred_element_type=jnp.float32)

    @pl.when(i == 0)
    def _():
        p_scr[...] = jnp.zeros_like(p_scr)

    p_scr[...] += part

    @pl.when(i == NBLK - 1)
    def _():
        z_ref[...] = jnp.dot(p_scr[...], wl_ref[...],
                             preferred_element_type=jnp.float32) + bl_ref[...]


_tc_final = pl.pallas_call(
    _tc_final_body,
    grid=(NBLK,),
    in_specs=[
        pl.BlockSpec((NC, R, D), lambda i: (0, i, 0)),
        pl.BlockSpec((R, D), lambda i: (i, 0)),
        pl.BlockSpec((R, 1), lambda i: (i, 0)),
        pl.BlockSpec((1, D), lambda i: (0, 0)),
        pl.BlockSpec((1, 1, R), lambda i: (i, 0, 0)),
        pl.BlockSpec((D, NCLS), lambda i: (0, 0)),
        pl.BlockSpec((1, NCLS), lambda i: (0, 0)),
    ],
    out_specs=[
        pl.BlockSpec((R, D), lambda i: (i, 0)),
        pl.BlockSpec((G, NCLS), lambda i: (0, 0)),
    ],
    out_shape=[
        jax.ShapeDtypeStruct((N, D), jnp.float32),
        jax.ShapeDtypeStruct((G, NCLS), jnp.float32),
    ],
    scratch_shapes=[pltpu.VMEM((G, D), jnp.float32)],
)


def kernel(x, edge_index, batch, W1, b1, W2, b2, W3, b3, W4, b4, Wl, bl):
    src = edge_index[0].astype(jnp.int32)
    dst = edge_index[1].astype(jnp.int32)
    pad = EP - E
    srcp = jnp.concatenate([src, jnp.zeros((pad,), jnp.int32)])
    pad_dst = PAD_NODE + jnp.arange(pad, dtype=jnp.int32) % (NP - N)
    dstp = jnp.concatenate([dst, pad_dst])
    src2 = srcp.reshape(EP // C, C)
    dst2 = dstp.reshape(EP // C, C)
    batch_row = batch.astype(jnp.int32).reshape(NBLK, 1, R)

    zrows = jnp.zeros((C, D), jnp.float32)
    zdeg = jnp.zeros((NPS, DW), jnp.float32)
    ones = jnp.ones((C, DW), jnp.float32)

    deg = _sc_degree(dst2, zdeg, ones).reshape(NC, NP, DW)
    degT = deg[:, :N, 0].T

    def prop(u):
        a = _sc_prop(u, src2, dst2, zrows)
        return a.reshape(NC, NP, D)

    u1, dis = _tc0(x, W1, degT)
    a1 = prop(u1)
    h1, u2 = _tc_mid(a1, u1, dis, b1.reshape(1, D), W2)
    a2 = prop(u2)
    h2, u3 = _tc_mid(a2, u2, dis, b2.reshape(1, D), W3)
    a3 = prop(u3)
    h3, u4 = _tc_mid(a3, u3, dis, b3.reshape(1, D), W4)
    a4 = prop(u4)
    h4, z = _tc_final(a4, u4, dis, b4.reshape(1, D), batch_row, Wl,
                      bl.reshape(1, NCLS))
    return (h1, h2, h3, h4, z)


# speedup vs baseline: 2.9937x; 2.9487x over previous
import functools

import jax
import jax.numpy as jnp
from jax import lax
from jax.experimental import pallas as pl
from jax.experimental.pallas import tpu as pltpu
from jax.experimental.pallas import tpu_sc as plsc

N = 10000
D = 128
E = 320000
EP = 327680
G = 128
NCLS = 10

NC, NS = 2, 16
NW = NC * NS
C = 80

NP = 10240
NPS = NP // NS
PAD_NODE = N
DW = 16

_mesh = plsc.VectorSubcoreMesh(core_axis_name="c", subcore_axis_name="s")


NCHUNK_DEG = EP // NW // C
NB_DEG = 8
NK_DEG = NCHUNK_DEG // NB_DEG


@functools.partial(
    pl.kernel,
    out_type=jax.ShapeDtypeStruct((NC * NP, DW), jnp.float32),
    mesh=_mesh,
    scratch_types=[
        pltpu.VMEM((NCHUNK_DEG, C), jnp.int32),
        pltpu.VMEM((C, DW), jnp.float32),
        pltpu.VMEM((NPS, DW), jnp.float32),
        pltpu.VMEM_SHARED((NP, DW), jnp.float32),
        [pltpu.SemaphoreType.DMA] * NB_DEG,
    ],
    compiler_params=pltpu.CompilerParams(use_tc_tiling_on_sc=False),
)
def _sc_degree(dst_hbm, zeros_hbm, ones_hbm, out_hbm, didx, ones, bounce, acc,
               sems):
    c = lax.axis_index("c")
    s = lax.axis_index("s")
    wid = s * NC + c
    pltpu.sync_copy(dst_hbm.at[pl.ds(wid * NCHUNK_DEG, NCHUNK_DEG)], didx)
    pltpu.sync_copy(ones_hbm, ones)
    pltpu.sync_copy(zeros_hbm, bounce)
    pltpu.sync_copy(bounce, acc.at[pl.ds(s * NPS, NPS)])
    plsc.subcore_barrier()

    def body(k, carry):
        for b in range(NB_DEG):
            j = k * NB_DEG + b

            @pl.when(k > 0)
            def _():
                pltpu.make_async_copy(ones, acc.at[didx.at[j - NB_DEG]],
                                      sems[b]).wait()

            pltpu.async_copy(ones, acc.at[didx.at[j]], sems[b], add=True)
        return carry

    lax.fori_loop(0, NK_DEG, body, 0)
    for b in range(NB_DEG):
        j = (NK_DEG - 1) * NB_DEG + b
        pltpu.make_async_copy(ones, acc.at[didx.at[j]], sems[b]).wait()
    plsc.subcore_barrier()
    pltpu.sync_copy(acc.at[pl.ds(s * NPS, NPS)], bounce)
    pltpu.sync_copy(bounce, out_hbm.at[pl.ds(c * NP + s * NPS, NPS)])


NCHUNK = EP // NW // C
NBG = 4
NGRP = NCHUNK // NBG
NPAIR = NGRP // 2
NQ = NPS // C


@functools.partial(
    pl.kernel,
    out_type=jax.ShapeDtypeStruct((NC * NP, D), jnp.float32),
    mesh=_mesh,
    scratch_types=[
        [pltpu.VMEM((NBG, C), jnp.int32)] * 2,
        [pltpu.VMEM((NBG, C), jnp.int32)] * 2,
        [pltpu.VMEM((C, D), jnp.float32)] * NBG,
        pltpu.VMEM_SHARED((NP, D), jnp.float32),
        [pltpu.SemaphoreType.DMA] * 2,
        [pltpu.SemaphoreType.DMA] * NBG,
        [pltpu.SemaphoreType.DMA] * NBG,
    ],
)
def _sc_prop(u_hbm, src_hbm, dst_hbm, zeros_hbm, out_hbm,
             sidx, didx, rows, acc, isem, gsem, ssem):
    c = lax.axis_index("c")
    s = lax.axis_index("s")
    w = c * NS + s
    gbase = w * NGRP

    def idx_load(g, p):
        pltpu.async_copy(src_hbm.at[pl.ds((gbase + g) * NBG, NBG)], sidx[p],
                         isem[p])
        pltpu.async_copy(dst_hbm.at[pl.ds((gbase + g) * NBG, NBG)], didx[p],
                         isem[p])

    def idx_wait(g, p):
        pltpu.make_async_copy(src_hbm.at[pl.ds((gbase + g) * NBG, NBG)],
                              sidx[p], isem[p]).wait()
        pltpu.make_async_copy(dst_hbm.at[pl.ds((gbase + g) * NBG, NBG)],
                              didx[p], isem[p]).wait()

    def gather(b, p):
        pltpu.async_copy(u_hbm.at[sidx[p].at[b]], rows[b], gsem[b])

    def gather_wait(b, p):
        pltpu.make_async_copy(u_hbm.at[sidx[p].at[b]], rows[b],
                              gsem[b]).wait()

    def scatter(b, p):
        pltpu.async_copy(rows[b], acc.at[didx[p].at[b]], ssem[b], add=True)

    def scatter_wait(b, p):
        pltpu.make_async_copy(rows[b], acc.at[didx[p].at[b]], ssem[b]).wait()

    idx_load(0, 0)
    pltpu.sync_copy(zeros_hbm, rows[0])
    zdescs = [
        pltpu.async_copy(rows[0], acc.at[pl.ds(s * NPS + q * C, C)],
                         ssem[q % NBG])
        for q in range(NQ)
    ]
    for dsc in zdescs:
        dsc.wait()
    plsc.subcore_barrier()
    idx_wait(0, 0)
    idx_load(1, 1)
    for b in range(NBG):
        gather(b, 0)

    def body(p, carry):
        for b in range(NBG):
            gather_wait(b, 0)
            scatter(b, 0)

        @pl.when(p < NPAIR - 1)
        def _():
            idx_wait(2 * p + 1, 1)

        for b in range(NBG):
            scatter_wait(b, 0)

            @pl.when(p < NPAIR - 1)
            def _():
                gather(b, 1)

        @pl.when(p < NPAIR - 1)
        def _():
            idx_load(2 * p + 2, 0)

        @pl.when(p < NPAIR - 1)
        def _():
            for b in range(NBG):
                gather_wait(b, 1)
                scatter(b, 1)
            idx_wait(2 * p + 2, 0)
            for b in range(NBG):
                scatter_wait(b, 1)
                gather(b, 0)
            idx_load(2 * p + 3, 1)

        return carry

    lax.fori_loop(0, NPAIR, body, 0)
    idx_wait(2 * NPAIR - 1, 1)
    for b in range(NBG):
        gather(b, 1)
    for b in range(NBG):
        gather_wait(b, 1)
        scatter(b, 1)
    for b in range(NBG):
        scatter_wait(b, 1)
    plsc.subcore_barrier()
    odescs = []
    for q in range(NQ):
        if q >= NBG:
            odescs[q - NBG].wait()
        pltpu.sync_copy(acc.at[pl.ds(s * NPS + q * C, C)], rows[q % NBG])
        odescs.append(pltpu.async_copy(
            rows[q % NBG],
            out_hbm.at[pl.ds(c * NP + s * NPS + q * C, C)], gsem[q % NBG]))
    for dsc in odescs[NQ - NBG:]:
        dsc.wait()


R = 400
NBLK = N // R


def _tc0_body(x_ref, w_ref, deg_ref, u_ref, dis_ref):
    d = deg_ref[:, 0:1] + deg_ref[:, 1:2] + 1.0
    dis = lax.rsqrt(d)
    dis_ref[...] = dis
    u_ref[...] = jnp.dot(x_ref[...], w_ref[...],
                         preferred_element_type=jnp.float32) * dis


_tc0 = pl.pallas_call(
    _tc0_body,
    grid=(NBLK,),
    in_specs=[
        pl.BlockSpec((R, D), lambda i: (i, 0)),
        pl.BlockSpec((D, D), lambda i: (0, 0)),
        pl.BlockSpec((R, 2), lambda i: (i, 0)),
    ],
    out_specs=[
        pl.BlockSpec((R, D), lambda i: (i, 0)),
        pl.BlockSpec((R, 1), lambda i: (i, 0)),
    ],
    out_shape=[
        jax.ShapeDtypeStruct((N, D), jnp.float32),
        jax.ShapeDtypeStruct((N, 1), jnp.float32),
    ],
)


def _tc_mid_body(a_ref, u_ref, dis_ref, b_ref, w_ref, h_ref, un_ref):
    dis = dis_ref[...]
    agg = (a_ref[0] + a_ref[1] + u_ref[...]) * dis + b_ref[...]
    h = jnp.maximum(agg, 0.0)
    h_ref[...] = h
    un_ref[...] = jnp.dot(h, w_ref[...],
                          preferred_element_type=jnp.float32) * dis


_tc_mid = pl.pallas_call(
    _tc_mid_body,
    grid=(NBLK,),
    in_specs=[
        pl.BlockSpec((NC, R, D), lambda i: (0, i, 0)),
        pl.BlockSpec((R, D), lambda i: (i, 0)),
        pl.BlockSpec((R, 1), lambda i: (i, 0)),
        pl.BlockSpec((1, D), lambda i: (0, 0)),
        pl.BlockSpec((D, D), lambda i: (0, 0)),
    ],
    out_specs=[
        pl.BlockSpec((R, D), lambda i: (i, 0)),
        pl.BlockSpec((R, D), lambda i: (i, 0)),
    ],
    out_shape=[
        jax.ShapeDtypeStruct((N, D), jnp.float32),
        jax.ShapeDtypeStruct((N, D), jnp.float32),
    ],
)


def _tc_final_body(a_ref, u_ref, dis_ref, b_ref, batch_ref, wl_ref, bl_ref,
                   h_ref, z_ref, p_scr):
    i = pl.program_id(0)
    agg = (a_ref[0] + a_ref[1] + u_ref[...]) * dis_ref[...] + b_ref[...]
    h = jnp.maximum(agg, 0.0)
    h_ref[...] = h
    onehot = (lax.broadcasted_iota(jnp.int32, (G, R), 0)
              == batch_ref[0]).astype(jnp.float32)
    part = jax.lax.dot_general(onehot, h, (((1,), (0,)), ((), ())),
                               preferred_element_type=jnp.float32)

    @pl.when(i == 0)
    def _():
        p_scr[...] = jnp.zeros_like(p_scr)

    p_scr[...] += part

    @pl.when(i == NBLK - 1)
    def _():
        z_ref[...] = jnp.dot(p_scr[...], wl_ref[...],
                             preferred_element_type=jnp.float32) + bl_ref[...]


_tc_final = pl.pallas_call(
    _tc_final_body,
    grid=(NBLK,),
    in_specs=[
        pl.BlockSpec((NC, R, D), lambda i: (0, i, 0)),
        pl.BlockSpec((R, D), lambda i: (i, 0)),
        pl.BlockSpec((R, 1), lambda i: (i, 0)),
        pl.BlockSpec((1, D), lambda i: (0, 0)),
        pl.BlockSpec((1, 1, R), lambda i: (i, 0, 0)),
        pl.BlockSpec((D, NCLS), lambda i: (0, 0)),
        pl.BlockSpec((1, NCLS), lambda i: (0, 0)),
    ],
    out_specs=[
        pl.BlockSpec((R, D), lambda i: (i, 0)),
        pl.BlockSpec((G, NCLS), lambda i: (0, 0)),
    ],
    out_shape=[
        jax.ShapeDtypeStruct((N, D), jnp.float32),
        jax.ShapeDtypeStruct((G, NCLS), jnp.float32),
    ],
    scratch_shapes=[pltpu.VMEM((G, D), jnp.float32)],
)


def kernel(x, edge_index, batch, W1, b1, W2, b2, W3, b3, W4, b4, Wl, bl):
    src = edge_index[0].astype(jnp.int32)
    dst = edge_index[1].astype(jnp.int32)
    pad = EP - E
    pad_idx = jnp.arange(pad, dtype=jnp.int32)
    srcp = jnp.concatenate([src, pad_idx % N])
    dstp = jnp.concatenate([dst, PAD_NODE + pad_idx % (NP - N)])
    src2 = srcp.reshape(EP // C, C)
    dst2 = dstp.reshape(EP // C, C)
    batch_row = batch.astype(jnp.int32).reshape(NBLK, 1, R)

    zrows = jnp.zeros((C, D), jnp.float32)
    zdeg = jnp.zeros((NPS, DW), jnp.float32)
    ones = jnp.ones((C, DW), jnp.float32)

    deg = _sc_degree(dst2, zdeg, ones).reshape(NC, NP, DW)
    degT = deg[:, :N, 0].T

    def prop(u):
        a = _sc_prop(u, src2, dst2, zrows)
        return a.reshape(NC, NP, D)

    u1, dis = _tc0(x, W1, degT)
    a1 = prop(u1)
    h1, u2 = _tc_mid(a1, u1, dis, b1.reshape(1, D), W2)
    a2 = prop(u2)
    h2, u3 = _tc_mid(a2, u2, dis, b2.reshape(1, D), W3)
    a3 = prop(u3)
    h3, u4 = _tc_mid(a3, u3, dis, b3.reshape(1, D), W4)
    a4 = prop(u4)
    h4, z = _tc_final(a4, u4, dis, b4.reshape(1, D), batch_row, Wl,
                      bl.reshape(1, NCLS))
    return (h1, h2, h3, h4, z)


# speedup vs baseline: 3.5191x; 1.1755x over previous
import functools

import jax
import jax.numpy as jnp
from jax import lax
from jax.experimental import pallas as pl
from jax.experimental.pallas import tpu as pltpu
from jax.experimental.pallas import tpu_sc as plsc

N = 10000
D = 128
E = 320000
EP = 327680
G = 128
NCLS = 10

NC, NS = 2, 16
NW = NC * NS
C = 80

NP = 10240
NPS = NP // NS
PAD_NODE = N
DW = 16

_mesh = plsc.VectorSubcoreMesh(core_axis_name="c", subcore_axis_name="s")


NCHUNK_DEG = EP // NW // C
NB_DEG = 8
NK_DEG = NCHUNK_DEG // NB_DEG


@functools.partial(
    pl.kernel,
    out_type=jax.ShapeDtypeStruct((NC * NP, DW), jnp.float32),
    mesh=_mesh,
    scratch_types=[
        pltpu.VMEM((NCHUNK_DEG, C), jnp.int32),
        pltpu.VMEM((C, DW), jnp.float32),
        pltpu.VMEM((NPS, DW), jnp.float32),
        pltpu.VMEM_SHARED((NP, DW), jnp.float32),
        [pltpu.SemaphoreType.DMA] * NB_DEG,
    ],
    compiler_params=pltpu.CompilerParams(use_tc_tiling_on_sc=False),
)
def _sc_degree(dst_hbm, zeros_hbm, ones_hbm, out_hbm, didx, ones, bounce, acc,
               sems):
    c = lax.axis_index("c")
    s = lax.axis_index("s")
    wid = s * NC + c
    pltpu.sync_copy(dst_hbm.at[pl.ds(wid * NCHUNK_DEG, NCHUNK_DEG)], didx)
    pltpu.sync_copy(ones_hbm, ones)
    pltpu.sync_copy(zeros_hbm, bounce)
    pltpu.sync_copy(bounce, acc.at[pl.ds(s * NPS, NPS)])
    plsc.subcore_barrier()

    def body(k, carry):
        for b in range(NB_DEG):
            j = k * NB_DEG + b

            @pl.when(k > 0)
            def _():
                pltpu.make_async_copy(ones, acc.at[didx.at[j - NB_DEG]],
                                      sems[b]).wait()

            pltpu.async_copy(ones, acc.at[didx.at[j]], sems[b], add=True)
        return carry

    lax.fori_loop(0, NK_DEG, body, 0)
    for b in range(NB_DEG):
        j = (NK_DEG - 1) * NB_DEG + b
        pltpu.make_async_copy(ones, acc.at[didx.at[j]], sems[b]).wait()
    plsc.subcore_barrier()
    pltpu.sync_copy(acc.at[pl.ds(s * NPS, NPS)], bounce)
    pltpu.sync_copy(bounce, out_hbm.at[pl.ds(c * NP + s * NPS, NPS)])


NCHUNK = EP // NW // C
NBG = 4
NGRP = NCHUNK // NBG
NPAIR = NGRP // 2
NQ = NPS // C


@functools.partial(
    pl.kernel,
    out_type=jax.ShapeDtypeStruct((NC * NP, D), jnp.float32),
    mesh=_mesh,
    scratch_types=[
        [pltpu.VMEM((NBG, C), jnp.int32)] * 2,
        [pltpu.VMEM((NBG, C), jnp.int32)] * 2,
        [pltpu.VMEM((C, D), jnp.float32)] * NBG,
        pltpu.VMEM_SHARED((NP, D), jnp.float32),
        [pltpu.SemaphoreType.DMA] * 2,
        [pltpu.SemaphoreType.DMA] * NBG,
        [pltpu.SemaphoreType.DMA] * NBG,
    ],
)
def _sc_prop(u_hbm, src_hbm, dst_hbm, zeros_hbm, out_hbm,
             sidx, didx, rows, acc, isem, gsem, ssem):
    c = lax.axis_index("c")
    s = lax.axis_index("s")
    w = c * NS + s
    gbase = w * NGRP

    def idx_load(g, p):
        pltpu.async_copy(src_hbm.at[pl.ds((gbase + g) * NBG, NBG)], sidx[p],
                         isem[p])
        pltpu.async_copy(dst_hbm.at[pl.ds((gbase + g) * NBG, NBG)], didx[p],
                         isem[p])

    def idx_wait(g, p):
        pltpu.make_async_copy(src_hbm.at[pl.ds((gbase + g) * NBG, NBG)],
                              sidx[p], isem[p]).wait()
        pltpu.make_async_copy(dst_hbm.at[pl.ds((gbase + g) * NBG, NBG)],
                              didx[p], isem[p]).wait()

    def gather(b, p):
        pltpu.async_copy(u_hbm.at[sidx[p].at[b]], rows[b], gsem[b])

    def gather_wait(b, p):
        pltpu.make_async_copy(u_hbm.at[sidx[p].at[b]], rows[b],
                              gsem[b]).wait()

    def scatter(b, p):
        pltpu.async_copy(rows[b], acc.at[didx[p].at[b]], ssem[b], add=True)

    def scatter_wait(b, p):
        pltpu.make_async_copy(rows[b], acc.at[didx[p].at[b]], ssem[b]).wait()

    idx_load(0, 0)
    pltpu.sync_copy(zeros_hbm, rows[0])
    zdescs = [
        pltpu.async_copy(rows[0], acc.at[pl.ds(s * NPS + q * C, C)],
                         ssem[q % NBG])
        for q in range(NQ)
    ]
    for dsc in zdescs:
        dsc.wait()
    plsc.subcore_barrier()
    idx_wait(0, 0)
    idx_load(1, 1)
    for b in range(NBG):
        gather(b, 0)

    def body(p, carry):
        for b in range(NBG):
            gather_wait(b, 0)
            scatter(b, 0)

        @pl.when(p < NPAIR - 1)
        def _():
            idx_wait(2 * p + 1, 1)

        for b in range(NBG):
            scatter_wait(b, 0)

            @pl.when(p < NPAIR - 1)
            def _():
                gather(b, 1)

        @pl.when(p < NPAIR - 1)
        def _():
            idx_load(2 * p + 2, 0)

        @pl.when(p < NPAIR - 1)
        def _():
            for b in range(NBG):
                gather_wait(b, 1)
                scatter(b, 1)
            idx_wait(2 * p + 2, 0)
            for b in range(NBG):
                scatter_wait(b, 1)
                gather(b, 0)
            idx_load(2 * p + 3, 1)

        return carry

    lax.fori_loop(0, NPAIR, body, 0)
    idx_wait(2 * NPAIR - 1, 1)
    for b in range(NBG):
        gather(b, 1)
    for b in range(NBG):
        gather_wait(b, 1)
        scatter(b, 1)
    for b in range(NBG):
        scatter_wait(b, 1)
    plsc.subcore_barrier()
    odescs = []
    for q in range(NQ):
        if q >= NBG:
            odescs[q - NBG].wait()
        pltpu.sync_copy(acc.at[pl.ds(s * NPS + q * C, C)], rows[q % NBG])
        odescs.append(pltpu.async_copy(
            rows[q % NBG],
            out_hbm.at[pl.ds(c * NP + s * NPS + q * C, C)], gsem[q % NBG]))
    for dsc in odescs[NQ - NBG:]:
        dsc.wait()


R = 1000
NBLK = N // R


def _tc0_body(x_ref, w_ref, deg0_ref, deg1_ref, u_ref, dis_ref):
    d = deg0_ref[...] + deg1_ref[...] + 1.0
    dis = lax.rsqrt(d)
    dis_ref[...] = dis
    u_ref[...] = jnp.dot(x_ref[...], w_ref[...],
                         preferred_element_type=jnp.float32) * dis


_tc0 = pl.pallas_call(
    _tc0_body,
    grid=(NBLK,),
    in_specs=[
        pl.BlockSpec((R, D), lambda i: (i, 0)),
        pl.BlockSpec((D, D), lambda i: (0, 0)),
        pl.BlockSpec((R, 1), lambda i: (i, 0)),
        pl.BlockSpec((R, 1), lambda i: (i, 0)),
    ],
    out_specs=[
        pl.BlockSpec((R, D), lambda i: (i, 0)),
        pl.BlockSpec((R, 1), lambda i: (i, 0)),
    ],
    out_shape=[
        jax.ShapeDtypeStruct((N, D), jnp.float32),
        jax.ShapeDtypeStruct((N, 1), jnp.float32),
    ],
)


def _tc_mid_body(a_ref, u_ref, dis_ref, b_ref, w_ref, h_ref, un_ref):
    dis = dis_ref[...]
    agg = (a_ref[0] + a_ref[1] + u_ref[...]) * dis + b_ref[...]
    h = jnp.maximum(agg, 0.0)
    h_ref[...] = h
    un_ref[...] = jnp.dot(h, w_ref[...],
                          preferred_element_type=jnp.float32) * dis


_tc_mid = pl.pallas_call(
    _tc_mid_body,
    grid=(NBLK,),
    in_specs=[
        pl.BlockSpec((NC, R, D), lambda i: (0, i, 0)),
        pl.BlockSpec((R, D), lambda i: (i, 0)),
        pl.BlockSpec((R, 1), lambda i: (i, 0)),
        pl.BlockSpec((1, D), lambda i: (0, 0)),
        pl.BlockSpec((D, D), lambda i: (0, 0)),
    ],
    out_specs=[
        pl.BlockSpec((R, D), lambda i: (i, 0)),
        pl.BlockSpec((R, D), lambda i: (i, 0)),
    ],
    out_shape=[
        jax.ShapeDtypeStruct((N, D), jnp.float32),
        jax.ShapeDtypeStruct((N, D), jnp.float32),
    ],
)


def _tc_final_body(a_ref, u_ref, dis_ref, b_ref, batch_ref, wl_ref, bl_ref,
                   h_ref, z_ref, p_scr):
    i = pl.program_id(0)
    agg = (a_ref[0] + a_ref[1] + u_ref[...]) * dis_ref[...] + b_ref[...]
    h = jnp.maximum(agg, 0.0)
    h_ref[...] = h
    onehot = (lax.broadcasted_iota(jnp.int32, (G, R), 0)
              == batch_ref[0]).astype(jnp.float32)
    part = jax.lax.dot_general(onehot, h, (((1,), (0,)), ((), ())),
                               preferred_element_type=jnp.float32)

    @pl.when(i == 0)
    def _():
        p_scr[...] = jnp.zeros_like(p_scr)

    p_scr[...] += part

    @pl.when(i == NBLK - 1)
    def _():
        z_ref[...] = jnp.dot(p_scr[...], wl_ref[...],
                             preferred_element_type=jnp.float32) + bl_ref[...]


_tc_final = pl.pallas_call(
    _tc_final_body,
    grid=(NBLK,),
    in_specs=[
        pl.BlockSpec((NC, R, D), lambda i: (0, i, 0)),
        pl.BlockSpec((R, D), lambda i: (i, 0)),
        pl.BlockSpec((R, 1), lambda i: (i, 0)),
        pl.BlockSpec((1, D), lambda i: (0, 0)),
        pl.BlockSpec((1, 1, R), lambda i: (i, 0, 0)),
        pl.BlockSpec((D, NCLS), lambda i: (0, 0)),
        pl.BlockSpec((1, NCLS), lambda i: (0, 0)),
    ],
    out_specs=[
        pl.BlockSpec((R, D), lambda i: (i, 0)),
        pl.BlockSpec((G, NCLS), lambda i: (0, 0)),
    ],
    out_shape=[
        jax.ShapeDtypeStruct((N, D), jnp.float32),
        jax.ShapeDtypeStruct((G, NCLS), jnp.float32),
    ],
    scratch_shapes=[pltpu.VMEM((G, D), jnp.float32)],
)


def kernel(x, edge_index, batch, W1, b1, W2, b2, W3, b3, W4, b4, Wl, bl):
    src = edge_index[0].astype(jnp.int32)
    dst = edge_index[1].astype(jnp.int32)
    pad = EP - E
    pad_idx = jnp.arange(pad, dtype=jnp.int32)
    srcp = jnp.concatenate([src, pad_idx % N])
    dstp = jnp.concatenate([dst, PAD_NODE + pad_idx % (NP - N)])
    src2 = srcp.reshape(EP // C, C)
    dst2 = dstp.reshape(EP // C, C)
    batch_row = batch.astype(jnp.int32).reshape(NBLK, 1, R)

    zrows = jnp.zeros((C, D), jnp.float32)
    zdeg = jnp.zeros((NPS, DW), jnp.float32)
    ones = jnp.ones((C, DW), jnp.float32)

    deg = _sc_degree(dst2, zdeg, ones)
    deg0 = lax.slice(deg, (0, 0), (N, 1))
    deg1 = lax.slice(deg, (NP, 0), (NP + N, 1))

    def prop(u):
        a = _sc_prop(u, src2, dst2, zrows)
        return a.reshape(NC, NP, D)

    u1, dis = _tc0(x, W1, deg0, deg1)
    a1 = prop(u1)
    h1, u2 = _tc_mid(a1, u1, dis, b1.reshape(1, D), W2)
    a2 = prop(u2)
    h2, u3 = _tc_mid(a2, u2, dis, b2.reshape(1, D), W3)
    a3 = prop(u3)
    h3, u4 = _tc_mid(a3, u3, dis, b3.reshape(1, D), W4)
    a4 = prop(u4)
    h4, z = _tc_final(a4, u4, dis, b4.reshape(1, D), batch_row, Wl,
                      bl.reshape(1, NCLS))
    return (h1, h2, h3, h4, z)


# speedup vs baseline: 3.5939x; 1.0212x over previous
import functools

import jax
import jax.numpy as jnp
from jax import lax
from jax.experimental import pallas as pl
from jax.experimental.pallas import tpu as pltpu
from jax.experimental.pallas import tpu_sc as plsc

N = 10000
D = 128
E = 320000
G = 128
NCLS = 10

NC, NS = 2, 16
NW = NC * NS
C = 80
NCHT = E // C
LASTC = NCHT - (NW - 1) * 128

NP = 10240
NPS = NP // NS
DW = 16

_mesh = plsc.VectorSubcoreMesh(core_axis_name="c", subcore_axis_name="s")


NCHUNK_DEG = 128
NB_DEG = 8
NK_DEG = NCHUNK_DEG // NB_DEG


@functools.partial(
    pl.kernel,
    out_type=jax.ShapeDtypeStruct((NC * NP, DW), jnp.float32),
    mesh=_mesh,
    scratch_types=[
        pltpu.VMEM((NCHUNK_DEG, C), jnp.int32),
        pltpu.VMEM((C, DW), jnp.float32),
        pltpu.VMEM((NPS, DW), jnp.float32),
        pltpu.VMEM_SHARED((NP, DW), jnp.float32),
        [pltpu.SemaphoreType.DMA] * NB_DEG,
    ],
    compiler_params=pltpu.CompilerParams(use_tc_tiling_on_sc=False),
)
def _sc_degree(dst_hbm, zeros_hbm, ones_hbm, out_hbm, didx, ones, bounce, acc,
               sems):
    c = lax.axis_index("c")
    s = lax.axis_index("s")
    wid = s * NC + c
    last = wid == NW - 1
    nk = jnp.where(last, LASTC // NB_DEG, NK_DEG)

    @pl.when(jnp.logical_not(last))
    def _():
        pltpu.sync_copy(dst_hbm.at[pl.ds(wid * NCHUNK_DEG, NCHUNK_DEG)], didx)

    @pl.when(last)
    def _():
        pltpu.sync_copy(dst_hbm.at[pl.ds(wid * NCHUNK_DEG, LASTC)],
                        didx.at[pl.ds(0, LASTC)])

    pltpu.sync_copy(ones_hbm, ones)
    pltpu.sync_copy(zeros_hbm, bounce)
    pltpu.sync_copy(bounce, acc.at[pl.ds(s * NPS, NPS)])
    plsc.subcore_barrier()

    def body(k, carry):
        for b in range(NB_DEG):
            j = k * NB_DEG + b

            @pl.when(k > 0)
            def _():
                pltpu.make_async_copy(ones, acc.at[didx.at[j - NB_DEG]],
                                      sems[b]).wait()

            pltpu.async_copy(ones, acc.at[didx.at[j]], sems[b], add=True)
        return carry

    lax.fori_loop(0, nk, body, 0)
    for b in range(NB_DEG):
        j = (nk - 1) * NB_DEG + b
        pltpu.make_async_copy(ones, acc.at[didx.at[j]], sems[b]).wait()
    plsc.subcore_barrier()
    pltpu.sync_copy(acc.at[pl.ds(s * NPS, NPS)], bounce)
    pltpu.sync_copy(bounce, out_hbm.at[pl.ds(c * NP + s * NPS, NPS)])


NCHUNK = 128
NBG = 4
NGRP = NCHUNK // NBG
NPAIR = NGRP // 2
NPAIR_LAST = LASTC // (2 * NBG)
NQ = NPS // C


@functools.partial(
    pl.kernel,
    out_type=jax.ShapeDtypeStruct((NC * NP, D), jnp.float32),
    mesh=_mesh,
    scratch_types=[
        [pltpu.VMEM((NBG, C), jnp.int32)] * 2,
        [pltpu.VMEM((NBG, C), jnp.int32)] * 2,
        [pltpu.VMEM((C, D), jnp.float32)] * NBG,
        pltpu.VMEM_SHARED((NP, D), jnp.float32),
        [pltpu.SemaphoreType.DMA] * 2,
        [pltpu.SemaphoreType.DMA] * NBG,
        [pltpu.SemaphoreType.DMA] * NBG,
    ],
)
def _sc_prop(u_hbm, src_hbm, dst_hbm, zeros_hbm, out_hbm,
             sidx, didx, rows, acc, isem, gsem, ssem):
    c = lax.axis_index("c")
    s = lax.axis_index("s")
    w = c * NS + s
    npair = jnp.where(w == NW - 1, NPAIR_LAST, NPAIR)
    gbase = w * NGRP

    def idx_load(g, p):
        pltpu.async_copy(src_hbm.at[pl.ds((gbase + g) * NBG, NBG)], sidx[p],
                         isem[p])
        pltpu.async_copy(dst_hbm.at[pl.ds((gbase + g) * NBG, NBG)], didx[p],
                         isem[p])

    def idx_wait(g, p):
        pltpu.make_async_copy(src_hbm.at[pl.ds((gbase + g) * NBG, NBG)],
                              sidx[p], isem[p]).wait()
        pltpu.make_async_copy(dst_hbm.at[pl.ds((gbase + g) * NBG, NBG)],
                              didx[p], isem[p]).wait()

    def gather(b, p):
        pltpu.async_copy(u_hbm.at[sidx[p].at[b]], rows[b], gsem[b])

    def gather_wait(b, p):
        pltpu.make_async_copy(u_hbm.at[sidx[p].at[b]], rows[b],
                              gsem[b]).wait()

    def scatter(b, p):
        pltpu.async_copy(rows[b], acc.at[didx[p].at[b]], ssem[b], add=True)

    def scatter_wait(b, p):
        pltpu.make_async_copy(rows[b], acc.at[didx[p].at[b]], ssem[b]).wait()

    idx_load(0, 0)
    pltpu.sync_copy(zeros_hbm, rows[0])
    zdescs = [
        pltpu.async_copy(rows[0], acc.at[pl.ds(s * NPS + q * C, C)],
                         ssem[q % NBG])
        for q in range(NQ)
    ]
    for dsc in zdescs:
        dsc.wait()
    plsc.subcore_barrier()
    idx_wait(0, 0)
    idx_load(1, 1)
    for b in range(NBG):
        gather(b, 0)

    def body(p, carry):
        for b in range(NBG):
            gather_wait(b, 0)
            scatter(b, 0)

        @pl.when(p < npair - 1)
        def _():
            idx_wait(2 * p + 1, 1)

        for b in range(NBG):
            scatter_wait(b, 0)

            @pl.when(p < npair - 1)
            def _():
                gather(b, 1)

        @pl.when(p < npair - 1)
        def _():
            idx_load(2 * p + 2, 0)

        @pl.when(p < npair - 1)
        def _():
            for b in range(NBG):
                gather_wait(b, 1)
                scatter(b, 1)
            idx_wait(2 * p + 2, 0)
            for b in range(NBG):
                scatter_wait(b, 1)
                gather(b, 0)
            idx_load(2 * p + 3, 1)

        return carry

    lax.fori_loop(0, npair, body, 0)
    idx_wait(2 * npair - 1, 1)
    for b in range(NBG):
        gather(b, 1)
    for b in range(NBG):
        gather_wait(b, 1)
        scatter(b, 1)
    for b in range(NBG):
        scatter_wait(b, 1)
    plsc.subcore_barrier()
    odescs = []
    for q in range(NQ):
        if q >= NBG:
            odescs[q - NBG].wait()
        pltpu.sync_copy(acc.at[pl.ds(s * NPS + q * C, C)], rows[q % NBG])
        odescs.append(pltpu.async_copy(
            rows[q % NBG],
            out_hbm.at[pl.ds(c * NP + s * NPS + q * C, C)], gsem[q % NBG]))
    for dsc in odescs[NQ - NBG:]:
        dsc.wait()


R = 2000
NBLK = N // R


def _tc0_body(x_ref, w_ref, deg0_ref, deg1_ref, u_ref, dis_ref):
    d = deg0_ref[...] + deg1_ref[...] + 1.0
    dis = lax.rsqrt(d)
    dis_ref[...] = dis
    u_ref[...] = jnp.dot(x_ref[...], w_ref[...],
                         preferred_element_type=jnp.float32) * dis


_tc0 = pl.pallas_call(
    _tc0_body,
    grid=(NBLK,),
    in_specs=[
        pl.BlockSpec((R, D), lambda i: (i, 0)),
        pl.BlockSpec((D, D), lambda i: (0, 0)),
        pl.BlockSpec((R, 1), lambda i: (i, 0)),
        pl.BlockSpec((R, 1), lambda i: (i, 0)),
    ],
    out_specs=[
        pl.BlockSpec((R, D), lambda i: (i, 0)),
        pl.BlockSpec((R, 1), lambda i: (i, 0)),
    ],
    out_shape=[
        jax.ShapeDtypeStruct((N, D), jnp.float32),
        jax.ShapeDtypeStruct((N, 1), jnp.float32),
    ],
)


def _tc_mid_body(a_ref, u_ref, dis_ref, b_ref, w_ref, h_ref, un_ref):
    dis = dis_ref[...]
    agg = (a_ref[0] + a_ref[1] + u_ref[...]) * dis + b_ref[...]
    h = jnp.maximum(agg, 0.0)
    h_ref[...] = h
    un_ref[...] = jnp.dot(h, w_ref[...],
                          preferred_element_type=jnp.float32) * dis


_tc_mid = pl.pallas_call(
    _tc_mid_body,
    grid=(NBLK,),
    in_specs=[
        pl.BlockSpec((NC, R, D), lambda i: (0, i, 0)),
        pl.BlockSpec((R, D), lambda i: (i, 0)),
        pl.BlockSpec((R, 1), lambda i: (i, 0)),
        pl.BlockSpec((1, D), lambda i: (0, 0)),
        pl.BlockSpec((D, D), lambda i: (0, 0)),
    ],
    out_specs=[
        pl.BlockSpec((R, D), lambda i: (i, 0)),
        pl.BlockSpec((R, D), lambda i: (i, 0)),
    ],
    out_shape=[
        jax.ShapeDtypeStruct((N, D), jnp.float32),
        jax.ShapeDtypeStruct((N, D), jnp.float32),
    ],
)


def _tc_final_body(a_ref, u_ref, dis_ref, b_ref, batch_ref, wl_ref, bl_ref,
                   h_ref, z_ref, p_scr):
    i = pl.program_id(0)
    agg = (a_ref[0] + a_ref[1] + u_ref[...]) * dis_ref[...] + b_ref[...]
    h = jnp.maximum(agg, 0.0)
    h_ref[...] = h
    onehot = (lax.broadcasted_iota(jnp.int32, (G, R), 0)
              == batch_ref[0]).astype(jnp.float32)
    part = jax.lax.dot_general(onehot, h, (((1,), (0,)), ((), ())),
                               preferred_element_type=jnp.float32)

    @pl.when(i == 0)
    def _():
        p_scr[...] = jnp.zeros_like(p_scr)

    p_scr[...] += part

    @pl.when(i == NBLK - 1)
    def _():
        z_ref[...] = jnp.dot(p_scr[...], wl_ref[...],
                             preferred_element_type=jnp.float32) + bl_ref[...]


_tc_final = pl.pallas_call(
    _tc_final_body,
    grid=(NBLK,),
    in_specs=[
        pl.BlockSpec((NC, R, D), lambda i: (0, i, 0)),
        pl.BlockSpec((R, D), lambda i: (i, 0)),
        pl.BlockSpec((R, 1), lambda i: (i, 0)),
        pl.BlockSpec((1, D), lambda i: (0, 0)),
        pl.BlockSpec((1, 1, R), lambda i: (i, 0, 0)),
        pl.BlockSpec((D, NCLS), lambda i: (0, 0)),
        pl.BlockSpec((1, NCLS), lambda i: (0, 0)),
    ],
    out_specs=[
        pl.BlockSpec((R, D), lambda i: (i, 0)),
        pl.BlockSpec((G, NCLS), lambda i: (0, 0)),
    ],
    out_shape=[
        jax.ShapeDtypeStruct((N, D), jnp.float32),
        jax.ShapeDtypeStruct((G, NCLS), jnp.float32),
    ],
    scratch_shapes=[pltpu.VMEM((G, D), jnp.float32)],
)


def kernel(x, edge_index, batch, W1, b1, W2, b2, W3, b3, W4, b4, Wl, bl):
    src2 = edge_index[0].astype(jnp.int32).reshape(NCHT, C)
    dst2 = edge_index[1].astype(jnp.int32).reshape(NCHT, C)
    batch_row = batch.astype(jnp.int32).reshape(NBLK, 1, R)

    zrows = jnp.zeros((C, D), jnp.float32)
    zdeg = jnp.zeros((NPS, DW), jnp.float32)
    ones = jnp.ones((C, DW), jnp.float32)

    deg = _sc_degree(dst2, zdeg, ones)
    deg0 = lax.slice(deg, (0, 0), (N, 1))
    deg1 = lax.slice(deg, (NP, 0), (NP + N, 1))

    def prop(u):
        a = _sc_prop(u, src2, dst2, zrows)
        return a.reshape(NC, NP, D)

    u1, dis = _tc0(x, W1, deg0, deg1)
    a1 = prop(u1)
    h1, u2 = _tc_mid(a1, u1, dis, b1.reshape(1, D), W2)
    a2 = prop(u2)
    h2, u3 = _tc_mid(a2, u2, dis, b2.reshape(1, D), W3)
    a3 = prop(u3)
    h3, u4 = _tc_mid(a3, u3, dis, b3.reshape(1, D), W4)
    a4 = prop(u4)
    h4, z = _tc_final(a4, u4, dis, b4.reshape(1, D), batch_row, Wl,
                      bl.reshape(1, NCLS))
    return (h1, h2, h3, h4, z)
